# baseline (speedup 1.0000x reference)
import jax
import jax.numpy as jnp
from jax import lax
from jax.experimental import pallas as pl
from jax.experimental.pallas import tpu as pltpu
from jax.experimental.pallas import tpu_sc as plsc

NC = 2
NS = 16
NW = NC * NS
L = 16

B = 16384
D = 64
CHUNK = B // NW
NB = 128
NJ = CHUNK // NB
GPB = NB // L


def _body(uidx_hbm, midx_hbm, utab_hbm, mtab_hbm, wb_hbm, out_hbm,
          uidx_v, midx_v, urows_v, mrows_v, wb_v, out_v, ptile_v, sem):
    wid = lax.axis_index("s") * NC + lax.axis_index("c")
    base = wid * CHUNK

    pltpu.sync_copy(uidx_hbm.at[wid], uidx_v)
    pltpu.sync_copy(midx_hbm.at[wid], midx_v)
    pltpu.sync_copy(wb_hbm, wb_v)

    wk = [wb_v[k, :] for k in range(D // L)]
    lane_iota = lax.iota(jnp.int32, L)
    bias_vec = wb_v[D // L, :]

    for j in range(NJ):
        def fire(g, carry):
            row0 = g * L
            uvec = uidx_v[j, pl.ds(row0, L)]
            mvec = midx_v[j, pl.ds(row0, L)]
            for e in range(L):
                pltpu.async_copy(
                    utab_hbm.at[uvec[e]], urows_v.at[row0 + e], sem)
                pltpu.async_copy(
                    mtab_hbm.at[mvec[e]], mrows_v.at[row0 + e], sem)
            return carry

        lax.fori_loop(0, GPB, fire, 0)
        pltpu.make_async_copy(utab_hbm.at[pl.ds(0, NB)], urows_v, sem).wait()
        pltpu.make_async_copy(mtab_hbm.at[pl.ds(0, NB)], mrows_v, sem).wait()

        def group(g, carry):
            row0 = g * L
            for r in range(L):
                row = row0 + r
                acc = None
                for k in range(D // L):
                    u = urows_v[row, pl.ds(L * k, L)]
                    m = mrows_v[row, pl.ds(L * k, L)]
                    t = (u * m) * wk[k]
                    acc = t if acc is None else acc + t
                ptile_v[r, :] = acc
            s = bias_vec
            for l in range(L):
                col = plsc.load_gather(
                    ptile_v, [lane_iota, jnp.full((L,), l, jnp.int32)])
                s = s + col
            out_v[pl.ds(j * NB + row0, L)] = s
            return carry

        lax.fori_loop(0, GPB, group, 0)

    pltpu.sync_copy(out_v, out_hbm.at[pl.ds(base, CHUNK)])


@jax.jit
def _run(uidx, midx, utab, mtab, wb):
    mesh = plsc.VectorSubcoreMesh(core_axis_name="c", subcore_axis_name="s")
    kern = pl.kernel(
        _body,
        out_type=jax.ShapeDtypeStruct((B,), jnp.float32),
        mesh=mesh,
        compiler_params=pltpu.CompilerParams(
            needs_layout_passes=False, use_tc_tiling_on_sc=True),
        scratch_types=[
            pltpu.VMEM((NJ, NB), jnp.int32),
            pltpu.VMEM((NJ, NB), jnp.int32),
            pltpu.VMEM((NB, D), jnp.float32),
            pltpu.VMEM((NB, D), jnp.float32),
            pltpu.VMEM((D // L + 1, L), jnp.float32),
            pltpu.VMEM((CHUNK,), jnp.float32),
            pltpu.VMEM((L, L), jnp.float32),
            pltpu.SemaphoreType.DMA,
        ],
    )
    return kern(uidx, midx, utab, mtab, wb)


def kernel(user, movie, user_table, movie_table, fc_w, fc_b):
    uidx = user.astype(jnp.int32).reshape(NW, NJ, NB)
    midx = movie.astype(jnp.int32).reshape(NW, NJ, NB)
    wb = jnp.concatenate(
        [fc_w.reshape(D), jnp.broadcast_to(fc_b.reshape(1), (L,))]
    ).astype(jnp.float32).reshape(D // L + 1, L)
    out = _run(uidx, midx, user_table, movie_table, wb)
    return out.reshape(B, 1)


# speedup vs baseline: 1.0146x; 1.0146x over previous
import jax
import jax.numpy as jnp
from jax import lax
from jax.experimental import pallas as pl
from jax.experimental.pallas import tpu as pltpu
from jax.experimental.pallas import tpu_sc as plsc

NC = 2
NS = 16
NW = NC * NS
L = 16

B = 16384
D = 64
CHUNK = B // NW
NB = 128
NJ = CHUNK // NB
GPB = NB // L


def _body(uidx_hbm, midx_hbm, utab_hbm, mtab_hbm, wb_hbm, out_hbm,
          uidx_v, midx_v, urows0_v, mrows0_v, urows1_v, mrows1_v,
          wb_v, out_v, ptile_v, sem0, sem1):
    wid = lax.axis_index("s") * NC + lax.axis_index("c")
    base = wid * CHUNK

    pltpu.sync_copy(uidx_hbm.at[wid], uidx_v)
    pltpu.sync_copy(midx_hbm.at[wid], midx_v)
    pltpu.sync_copy(wb_hbm, wb_v)

    ubufs = [urows0_v, urows1_v]
    mbufs = [mrows0_v, mrows1_v]
    sems = [sem0, sem1]

    wk = [wb_v[k, :] for k in range(D // L)]
    lane_iota = lax.iota(jnp.int32, L)
    bias_vec = wb_v[D // L, :]

    def fire(j, p):
        ub, mb, sm = ubufs[p], mbufs[p], sems[p]

        def fire_g(g, carry):
            row0 = g * L
            uvec = uidx_v[j, pl.ds(row0, L)]
            mvec = midx_v[j, pl.ds(row0, L)]
            for e in range(L):
                pltpu.async_copy(utab_hbm.at[uvec[e]], ub.at[row0 + e], sm)
                pltpu.async_copy(mtab_hbm.at[mvec[e]], mb.at[row0 + e], sm)
            return carry

        lax.fori_loop(0, GPB, fire_g, 0)

    def drain(p):
        pltpu.make_async_copy(
            utab_hbm.at[pl.ds(0, NB)], ubufs[p], sems[p]).wait()
        pltpu.make_async_copy(
            mtab_hbm.at[pl.ds(0, NB)], mbufs[p], sems[p]).wait()

    def compute(j, p):
        ub, mb = ubufs[p], mbufs[p]

        def group(g, carry):
            row0 = g * L
            for r in range(L):
                row = row0 + r
                acc = None
                for k in range(D // L):
                    u = ub[row, pl.ds(L * k, L)]
                    m = mb[row, pl.ds(L * k, L)]
                    t = (u * m) * wk[k]
                    acc = t if acc is None else acc + t
                ptile_v[r, :] = acc
            s = bias_vec
            for l in range(L):
                col = plsc.load_gather(
                    ptile_v, [lane_iota, jnp.full((L,), l, jnp.int32)])
                s = s + col
            out_v[pl.ds(j * NB + row0, L)] = s
            return carry

        lax.fori_loop(0, GPB, group, 0)

    fire(0, 0)
    for j in range(NJ):
        if j + 1 < NJ:
            fire(j + 1, (j + 1) % 2)
        drain(j % 2)
        compute(j, j % 2)

    pltpu.sync_copy(out_v, out_hbm.at[pl.ds(base, CHUNK)])


@jax.jit
def _run(uidx, midx, utab, mtab, wb):
    mesh = plsc.VectorSubcoreMesh(core_axis_name="c", subcore_axis_name="s")
    kern = pl.kernel(
        _body,
        out_type=jax.ShapeDtypeStruct((B,), jnp.float32),
        mesh=mesh,
        compiler_params=pltpu.CompilerParams(
            needs_layout_passes=False, use_tc_tiling_on_sc=True),
        scratch_types=[
            pltpu.VMEM((NJ, NB), jnp.int32),
            pltpu.VMEM((NJ, NB), jnp.int32),
            pltpu.VMEM((NB, D), jnp.float32),
            pltpu.VMEM((NB, D), jnp.float32),
            pltpu.VMEM((NB, D), jnp.float32),
            pltpu.VMEM((NB, D), jnp.float32),
            pltpu.VMEM((D // L + 1, L), jnp.float32),
            pltpu.VMEM((CHUNK,), jnp.float32),
            pltpu.VMEM((L, L), jnp.float32),
            pltpu.SemaphoreType.DMA,
            pltpu.SemaphoreType.DMA,
        ],
    )
    return kern(uidx, midx, utab, mtab, wb)


def kernel(user, movie, user_table, movie_table, fc_w, fc_b):
    uidx = user.astype(jnp.int32).reshape(NW, NJ, NB)
    midx = movie.astype(jnp.int32).reshape(NW, NJ, NB)
    wb = jnp.concatenate(
        [fc_w.reshape(D), jnp.broadcast_to(fc_b.reshape(1), (L,))]
    ).astype(jnp.float32).reshape(D // L + 1, L)
    out = _run(uidx, midx, user_table, movie_table, wb)
    return out.reshape(B, 1)
